# baseline (speedup 1.0000x reference)
import functools

import jax
import jax.numpy as jnp
from jax import lax
from jax.experimental import pallas as pl
from jax.experimental.pallas import tpu as pltpu
from jax.experimental.pallas import tpu_sc as plsc

_ROWS, _COLS, _K = 16384, 512, 128
_SC_ROWS = 7168
_TC_ROWS = _ROWS - _SC_ROWS
_TC_BLK = 1024
_NC, _NS = 2, 16
_NW = _NC * _NS
_RPW = _SC_ROWS // _NW
_CHUNK = 16
_UNROLL = 4
_NCHUNK = _RPW // _CHUNK
_NPAIR = _NCHUNK // 2
_L = 16


def _sc_gather_call(x, indices):
    mesh = plsc.VectorSubcoreMesh(core_axis_name="c", subcore_axis_name="s")

    @functools.partial(
        pl.kernel,
        mesh=mesh,
        out_type=jax.ShapeDtypeStruct((_SC_ROWS, _K), jnp.float32),
        scratch_types=[
            pltpu.VMEM((_K,), jnp.int32),
            pltpu.VMEM((2, _CHUNK, _COLS), jnp.float32),
            pltpu.VMEM((2, _CHUNK, _K), jnp.float32),
            pltpu.SemaphoreType.DMA,
            pltpu.SemaphoreType.DMA,
            pltpu.SemaphoreType.DMA,
            pltpu.SemaphoreType.DMA,
        ],
        compiler_params=pltpu.CompilerParams(needs_layout_passes=False),
    )
    def sc_gather(x_hbm, idx_hbm, out_hbm, idx_v, in_v, out_v,
                  si0, si1, so0, so1):
        wid = lax.axis_index("s") * _NC + lax.axis_index("c")
        base = wid * _RPW
        pltpu.sync_copy(idx_hbm, idx_v)
        idx_regs = [idx_v[pl.ds(k * _L, _L)] for k in range(_K // _L)]
        sin = [si0, si1]
        sout = [so0, so1]
        b_vecs = [jnp.full((_L,), b, jnp.int32) for b in range(2)]

        for b in range(2):
            pltpu.async_copy(
                x_hbm.at[pl.ds(base + b * _CHUNK, _CHUNK)], in_v.at[b], sin[b]
            )

        def pair_body(g, carry):
            for b in range(2):
                c = g * 2 + b
                r0 = base + c * _CHUNK
                pltpu.make_async_copy(
                    x_hbm.at[pl.ds(r0, _CHUNK)], in_v.at[b], sin[b]
                ).wait()

                @pl.when(g > 0)
                def _wait_prev_out():
                    pltpu.make_async_copy(
                        out_v.at[b], out_hbm.at[pl.ds(r0, _CHUNK)], sout[b]
                    ).wait()

                def row_body(rr, carry2):
                    for u in range(_UNROLL):
                        r = rr * _UNROLL + u
                        r_vec = jnp.full((_L,), r, jnp.int32)
                        for k in range(_K // _L):
                            out_v[b, r, pl.ds(k * _L, _L)] = plsc.load_gather(
                                in_v, [b_vecs[b], r_vec, idx_regs[k]]
                            )
                    return carry2

                lax.fori_loop(0, _CHUNK // _UNROLL, row_body, 0)
                pltpu.async_copy(
                    out_v.at[b], out_hbm.at[pl.ds(r0, _CHUNK)], sout[b]
                )

                @pl.when(g < _NPAIR - 1)
                def _start_next_in():
                    pltpu.async_copy(
                        x_hbm.at[pl.ds(r0 + 2 * _CHUNK, _CHUNK)],
                        in_v.at[b],
                        sin[b],
                    )

            return carry

        lax.fori_loop(0, _NPAIR, pair_body, 0)
        for b in range(2):
            r_last = base + (_NCHUNK - 2 + b) * _CHUNK
            pltpu.make_async_copy(
                out_v.at[b], out_hbm.at[pl.ds(r_last, _CHUNK)], sout[b]
            ).wait()

    return sc_gather(x, indices)


def _tc_body(idx_ref, x_ref, out_ref):
    col = lax.broadcasted_iota(jnp.int32, (_COLS, _K), 0)
    onehot = (col == idx_ref[0][None, :]).astype(jnp.float32)
    out_ref[...] = jnp.dot(
        x_ref[...], onehot, preferred_element_type=jnp.float32
    )


def _tc_gather_call(x, indices):
    grid = (_TC_ROWS // _TC_BLK,)
    return pl.pallas_call(
        _tc_body,
        grid=grid,
        in_specs=[
            pl.BlockSpec((1, _K), lambda i: (0, 0)),
            pl.BlockSpec((_TC_BLK, _COLS), lambda i: (i + _SC_ROWS // _TC_BLK, 0)),
        ],
        out_specs=pl.BlockSpec((_TC_BLK, _K), lambda i: (i, 0)),
        out_shape=jax.ShapeDtypeStruct((_TC_ROWS, _K), jnp.float32),
        compiler_params=pltpu.CompilerParams(
            dimension_semantics=("arbitrary",),
        ),
    )(indices.reshape(1, _K), x)


def kernel(x, indices):
    idx = indices.astype(jnp.int32)
    out_sc = _sc_gather_call(x, idx)
    out_tc = _tc_gather_call(x, idx)
    return jnp.concatenate([out_sc, out_tc], axis=0)


# speedup vs baseline: 1.0690x; 1.0690x over previous
import functools

import jax
import jax.numpy as jnp
from jax import lax
from jax.experimental import pallas as pl
from jax.experimental.pallas import tpu as pltpu
from jax.experimental.pallas import tpu_sc as plsc

_ROWS, _COLS, _K = 16384, 512, 128
_SC_ROWS = 6144
_TC_ROWS = _ROWS - _SC_ROWS
_TC_BLK = 1024
_NC, _NS = 2, 16
_NW = _NC * _NS
_RPW = _SC_ROWS // _NW
_CHUNK = 32
_UNROLL = 4
_NCHUNK = _RPW // _CHUNK
_NPAIR = _NCHUNK // 2
_L = 16


def _sc_gather_call(x, indices):
    mesh = plsc.VectorSubcoreMesh(core_axis_name="c", subcore_axis_name="s")

    @functools.partial(
        pl.kernel,
        mesh=mesh,
        out_type=jax.ShapeDtypeStruct((_SC_ROWS, _K), jnp.float32),
        scratch_types=[
            pltpu.VMEM((_K,), jnp.int32),
            pltpu.VMEM((2, _CHUNK, _COLS), jnp.float32),
            pltpu.VMEM((2, _CHUNK, _K), jnp.float32),
            pltpu.SemaphoreType.DMA,
            pltpu.SemaphoreType.DMA,
            pltpu.SemaphoreType.DMA,
            pltpu.SemaphoreType.DMA,
        ],
        compiler_params=pltpu.CompilerParams(needs_layout_passes=False),
    )
    def sc_gather(x_hbm, idx_hbm, out_hbm, idx_v, in_v, out_v,
                  si0, si1, so0, so1):
        wid = lax.axis_index("s") * _NC + lax.axis_index("c")
        base = wid * _RPW
        pltpu.sync_copy(idx_hbm, idx_v)
        idx_regs = [idx_v[pl.ds(k * _L, _L)] for k in range(_K // _L)]
        sin = [si0, si1]
        sout = [so0, so1]
        b_vecs = [jnp.full((_L,), b, jnp.int32) for b in range(2)]

        for b in range(2):
            pltpu.async_copy(
                x_hbm.at[pl.ds(base + b * _CHUNK, _CHUNK)], in_v.at[b], sin[b]
            )

        def pair_body(g, carry):
            for b in range(2):
                c = g * 2 + b
                r0 = base + c * _CHUNK
                pltpu.make_async_copy(
                    x_hbm.at[pl.ds(r0, _CHUNK)], in_v.at[b], sin[b]
                ).wait()

                @pl.when(g > 0)
                def _wait_prev_out():
                    pltpu.make_async_copy(
                        out_v.at[b], out_hbm.at[pl.ds(r0, _CHUNK)], sout[b]
                    ).wait()

                def row_body(rr, carry2):
                    for u in range(_UNROLL):
                        r = rr * _UNROLL + u
                        r_vec = jnp.full((_L,), r, jnp.int32)
                        for k in range(_K // _L):
                            out_v[b, r, pl.ds(k * _L, _L)] = plsc.load_gather(
                                in_v, [b_vecs[b], r_vec, idx_regs[k]]
                            )
                    return carry2

                lax.fori_loop(0, _CHUNK // _UNROLL, row_body, 0)
                pltpu.async_copy(
                    out_v.at[b], out_hbm.at[pl.ds(r0, _CHUNK)], sout[b]
                )

                @pl.when(g < _NPAIR - 1)
                def _start_next_in():
                    pltpu.async_copy(
                        x_hbm.at[pl.ds(r0 + 2 * _CHUNK, _CHUNK)],
                        in_v.at[b],
                        sin[b],
                    )

            return carry

        lax.fori_loop(0, _NPAIR, pair_body, 0)
        for b in range(2):
            r_last = base + (_NCHUNK - 2 + b) * _CHUNK
            pltpu.make_async_copy(
                out_v.at[b], out_hbm.at[pl.ds(r_last, _CHUNK)], sout[b]
            ).wait()

    return sc_gather(x, indices)


def _tc_body(idx_ref, x_ref, out_ref):
    col = lax.broadcasted_iota(jnp.int32, (_COLS, _K), 0)
    onehot = (col == idx_ref[0][None, :]).astype(jnp.float32)
    out_ref[...] = jnp.dot(
        x_ref[...], onehot, preferred_element_type=jnp.float32
    )


def _tc_gather_call(x, indices):
    grid = (_TC_ROWS // _TC_BLK,)
    return pl.pallas_call(
        _tc_body,
        grid=grid,
        in_specs=[
            pl.BlockSpec((1, _K), lambda i: (0, 0)),
            pl.BlockSpec((_TC_BLK, _COLS), lambda i: (i + _SC_ROWS // _TC_BLK, 0)),
        ],
        out_specs=pl.BlockSpec((_TC_BLK, _K), lambda i: (i, 0)),
        out_shape=jax.ShapeDtypeStruct((_TC_ROWS, _K), jnp.float32),
        compiler_params=pltpu.CompilerParams(
            dimension_semantics=("arbitrary",),
        ),
    )(indices.reshape(1, _K), x)


def kernel(x, indices):
    idx = indices.astype(jnp.int32)
    out_sc = _sc_gather_call(x, idx)
    out_tc = _tc_gather_call(x, idx)
    return jnp.concatenate([out_sc, out_tc], axis=0)
